# baseline (speedup 1.0000x reference)
import functools

import jax
import jax.numpy as jnp
from jax import lax
from jax.experimental import pallas as pl
from jax.experimental.pallas import tpu as pltpu
from jax.experimental.pallas import tpu_sc as plsc

N = 10000
E = 320000
D_IN = 128
HID = 128
OUT = 64

NC = 2
NS = 16
NW = NC * NS
CHUNK = 128
CPT = 80
NCHUNK = NW * CPT
E_PAD = NCHUNK * CHUNK - E
N_PAD = 10240
ROWS_PER_TILE = N_PAD // NS

_mesh = plsc.VectorSubcoreMesh(core_axis_name="c", subcore_axis_name="s")


def _zero_fill_vmem(buf, rows, width):
    zero16 = jnp.zeros((16,), jnp.float32)

    @pl.loop(0, rows)
    def _(i):
        @pl.loop(0, width // 16)
        def _(j):
            buf[i, pl.ds(j * 16, 16)] = zero16


def _zero_acc_slice(zeros_v, acc, s):
    @pl.loop(0, ROWS_PER_TILE // 16)
    def _(j):
        pltpu.sync_copy(zeros_v, acc.at[pl.ds(s * ROWS_PER_TILE + j * 16, 16)])


@functools.partial(
    pl.kernel,
    out_type=jax.ShapeDtypeStruct((NC, N_PAD, 16), jnp.float32),
    mesh=_mesh,
    scratch_types=[
        pltpu.VMEM((CPT, CHUNK), jnp.int32),
        pltpu.VMEM((CHUNK, 16), jnp.float32),
        pltpu.VMEM((16, 16), jnp.float32),
        pltpu.VMEM_SHARED((N_PAD, 16), jnp.float32),
    ],
)
def _deg_kernel(dst_hbm, out_hbm, dst_v, ones_v, zeros_v, acc):
    c = lax.axis_index("c")
    s = lax.axis_index("s")
    w = s * NC + c

    one16 = jnp.ones((16,), jnp.float32)

    @pl.loop(0, CHUNK)
    def _(i):
        ones_v[i, pl.ds(0, 16)] = one16

    _zero_fill_vmem(zeros_v, 16, 16)
    _zero_acc_slice(zeros_v, acc, s)
    pltpu.sync_copy(dst_hbm.at[pl.ds(w * CPT, CPT)], dst_v)

    plsc.subcore_barrier()

    @pl.loop(0, CPT)
    def _(j):
        pltpu.sync_copy(ones_v, acc.at[dst_v.at[j]], add=True)

    plsc.subcore_barrier()

    pltpu.sync_copy(
        acc.at[pl.ds(s * ROWS_PER_TILE, ROWS_PER_TILE)],
        out_hbm.at[c, pl.ds(s * ROWS_PER_TILE, ROWS_PER_TILE)],
    )


SLAB = 8
NSLAB = CPT // SLAB


@functools.partial(
    pl.kernel,
    out_type=jax.ShapeDtypeStruct((NC, N_PAD, HID), jnp.float32),
    mesh=_mesh,
    scratch_types=[
        pltpu.VMEM((2, SLAB, CHUNK), jnp.int32),
        pltpu.VMEM((2, SLAB, CHUNK), jnp.int32),
        pltpu.VMEM((CHUNK, HID), jnp.float32),
        pltpu.VMEM((CHUNK, HID), jnp.float32),
        pltpu.VMEM((16, HID), jnp.float32),
        pltpu.VMEM_SHARED((N_PAD, HID), jnp.float32),
        pltpu.SemaphoreType.DMA,
        pltpu.SemaphoreType.DMA,
        pltpu.SemaphoreType.DMA,
        pltpu.SemaphoreType.DMA,
    ],
)
def _scatter_kernel(src_hbm, dst_hbm, y_hbm, out_hbm,
                    src_v, dst_v, rows0, rows1, zeros_v, acc,
                    sem0, sem1, isem0, isem1):
    c = lax.axis_index("c")
    s = lax.axis_index("s")
    w = s * NC + c
    base = w * CPT

    _zero_fill_vmem(zeros_v, 16, HID)
    _zero_acc_slice(zeros_v, acc, s)

    isems = (isem0, isem1)
    rbufs = (rows0, rows1)
    gsems = (sem0, sem1)

    def idx_load(b, k):
        pltpu.async_copy(src_hbm.at[pl.ds(base + k * SLAB, SLAB)],
                         src_v.at[b], isems[b])
        pltpu.async_copy(dst_hbm.at[pl.ds(base + k * SLAB, SLAB)],
                         dst_v.at[b], isems[b])

    def idx_wait(b):
        pltpu.make_async_copy(src_hbm.at[pl.ds(base, SLAB)],
                              src_v.at[b], isems[b]).wait()
        pltpu.make_async_copy(dst_hbm.at[pl.ds(base, SLAB)],
                              dst_v.at[b], isems[b]).wait()

    idx_load(0, 0)

    plsc.subcore_barrier()

    def process_slab(b, k, prefetch_k):
        idx_wait(b)
        ob = 1 - b
        if prefetch_k is not None:
            idx_load(ob, prefetch_k)
        pltpu.async_copy(y_hbm.at[src_v.at[b, 0]], rbufs[0], gsems[0])
        for j in range(SLAB):
            rb = j % 2
            if j + 1 < SLAB:
                pltpu.async_copy(y_hbm.at[src_v.at[b, j + 1]],
                                 rbufs[1 - rb], gsems[1 - rb])
            pltpu.make_async_copy(y_hbm.at[src_v.at[b, 0]],
                                  rbufs[rb], gsems[rb]).wait()
            pltpu.sync_copy(rbufs[rb], acc.at[dst_v.at[b, j]], add=True)

    @pl.loop(0, NSLAB // 2)
    def _(t):
        k = 2 * t
        process_slab(0, k, k + 1)
        nxt = k + 2

        def load_next():
            idx_load(0, nxt)

        pl.when(nxt < NSLAB)(load_next)
        process_slab(1, k + 1, None)

    plsc.subcore_barrier()

    pltpu.sync_copy(
        acc.at[pl.ds(s * ROWS_PER_TILE, ROWS_PER_TILE)],
        out_hbm.at[c, pl.ds(s * ROWS_PER_TILE, ROWS_PER_TILE)],
    )


def _dinv_from_deg(degp_ref):
    deg = degp_ref[0, :N, 0:1] + degp_ref[1, :N, 0:1] + 1.0
    return lax.rsqrt(deg)


def _tc0_body(x_ref, w1_ref, xw_ref):
    xw_ref[...] = jnp.dot(x_ref[...], w1_ref[...],
                          preferred_element_type=jnp.float32)


def _tc1_body(xw_ref, degp_ref, y_ref):
    y_ref[...] = xw_ref[...] * _dinv_from_deg(degp_ref)


def _tc2_body(y_ref, aggp_ref, degp_ref, w2_ref, b1_ref, y2_ref):
    dinv = _dinv_from_deg(degp_ref)
    z = aggp_ref[0, :N, :] + aggp_ref[1, :N, :] + y_ref[...]
    h = jnp.maximum(dinv * z + b1_ref[...], 0.0)
    y2_ref[...] = jnp.dot(h, w2_ref[...], preferred_element_type=jnp.float32) * dinv


def _tc3_body(y_ref, aggp_ref, degp_ref, w3_ref, b2_ref, b3_ref, out_ref):
    dinv = _dinv_from_deg(degp_ref)
    z = aggp_ref[0, :N, :] + aggp_ref[1, :N, :] + y_ref[...]
    h = jnp.maximum(dinv * z + b2_ref[...], 0.0)
    logits = jnp.dot(h, w3_ref[...], preferred_element_type=jnp.float32) + b3_ref[...]
    m = jnp.max(logits, axis=1, keepdims=True)
    e = jnp.exp(logits - m)
    lse = jnp.log(jnp.sum(e, axis=1, keepdims=True)) + m
    out_ref[...] = logits - lse


def kernel(x, edge_index, W1, b1, W2, b2, W3, b3):
    pad_src = jnp.zeros((E_PAD,), jnp.int32)
    pad_dst = jnp.full((E_PAD,), N, jnp.int32)
    src = jnp.concatenate([edge_index[0], pad_src]).reshape(NCHUNK, CHUNK)
    dst = jnp.concatenate([edge_index[1], pad_dst]).reshape(NCHUNK, CHUNK)

    degp = _deg_kernel(dst)
    xw1 = pl.pallas_call(
        _tc0_body,
        out_shape=jax.ShapeDtypeStruct((N, D_IN), jnp.float32),
    )(x.astype(jnp.float32), W1)

    y1 = pl.pallas_call(
        _tc1_body,
        out_shape=jax.ShapeDtypeStruct((N, D_IN), jnp.float32),
    )(xw1, degp)

    agg1 = _scatter_kernel(src, dst, y1)

    y2 = pl.pallas_call(
        _tc2_body,
        out_shape=jax.ShapeDtypeStruct((N, HID), jnp.float32),
    )(y1, agg1, degp, W2, b1.reshape(1, HID))

    agg2 = _scatter_kernel(src, dst, y2)

    out = pl.pallas_call(
        _tc3_body,
        out_shape=jax.ShapeDtypeStruct((N, OUT), jnp.float32),
    )(y2, agg2, degp, W3, b2.reshape(1, HID), b3.reshape(1, OUT))

    return out


# speedup vs baseline: 3.0229x; 3.0229x over previous
import functools

import jax
import jax.numpy as jnp
from jax import lax
from jax.experimental import pallas as pl
from jax.experimental.pallas import tpu as pltpu
from jax.experimental.pallas import tpu_sc as plsc

N = 10000
E = 320000
D_IN = 128
HID = 128
OUT = 64

NC = 2
NS = 16
NW = NC * NS
CHUNK = 128
CPT = 80
NCHUNK = NW * CPT
E_PAD = NCHUNK * CHUNK - E
N_PAD = 10240
ROWS_PER_TILE = N_PAD // NS

_mesh = plsc.VectorSubcoreMesh(core_axis_name="c", subcore_axis_name="s")


def _zero_fill_vmem(buf, rows, width):
    zero16 = jnp.zeros((16,), jnp.float32)

    @pl.loop(0, rows)
    def _(i):
        @pl.loop(0, width // 16)
        def _(j):
            buf[i, pl.ds(j * 16, 16)] = zero16


def _zero_acc_slice(zeros_v, acc, s):
    @pl.loop(0, ROWS_PER_TILE // 16)
    def _(j):
        pltpu.sync_copy(zeros_v, acc.at[pl.ds(s * ROWS_PER_TILE + j * 16, 16)])


@functools.partial(
    pl.kernel,
    out_type=jax.ShapeDtypeStruct((NC, N_PAD, 16), jnp.float32),
    mesh=_mesh,
    scratch_types=[
        pltpu.VMEM((CPT, CHUNK), jnp.int32),
        pltpu.VMEM((CHUNK, 16), jnp.float32),
        pltpu.VMEM((16, 16), jnp.float32),
        pltpu.VMEM_SHARED((N_PAD, 16), jnp.float32),
    ],
)
def _deg_kernel(dst_hbm, out_hbm, dst_v, ones_v, zeros_v, acc):
    c = lax.axis_index("c")
    s = lax.axis_index("s")
    w = s * NC + c

    one16 = jnp.ones((16,), jnp.float32)

    @pl.loop(0, CHUNK)
    def _(i):
        ones_v[i, pl.ds(0, 16)] = one16

    _zero_fill_vmem(zeros_v, 16, 16)
    _zero_acc_slice(zeros_v, acc, s)
    pltpu.sync_copy(dst_hbm.at[pl.ds(w * CPT, CPT)], dst_v)

    plsc.subcore_barrier()

    @pl.loop(0, CPT)
    def _(j):
        pltpu.sync_copy(ones_v, acc.at[dst_v.at[j]], add=True)

    plsc.subcore_barrier()

    pltpu.sync_copy(
        acc.at[pl.ds(s * ROWS_PER_TILE, ROWS_PER_TILE)],
        out_hbm.at[c, pl.ds(s * ROWS_PER_TILE, ROWS_PER_TILE)],
    )


SLAB = 8
NSLAB = CPT // SLAB


@functools.partial(
    pl.kernel,
    out_type=jax.ShapeDtypeStruct((NC, N_PAD, HID), jnp.float32),
    mesh=_mesh,
    scratch_types=[
        pltpu.VMEM((2, SLAB, CHUNK), jnp.int32),
        pltpu.VMEM((2, SLAB, CHUNK), jnp.int32),
        pltpu.VMEM((CHUNK, HID), jnp.float32),
        pltpu.VMEM((CHUNK, HID), jnp.float32),
        pltpu.VMEM((16, HID), jnp.float32),
        pltpu.VMEM_SHARED((N_PAD, HID), jnp.float32),
        pltpu.SemaphoreType.DMA,
        pltpu.SemaphoreType.DMA,
        pltpu.SemaphoreType.DMA,
        pltpu.SemaphoreType.DMA,
    ],
)
def _scatter_kernel(src_hbm, dst_hbm, y_hbm, out_hbm,
                    src_v, dst_v, rows0, rows1, zeros_v, acc,
                    sem0, sem1, isem0, isem1):
    c = lax.axis_index("c")
    s = lax.axis_index("s")
    w = s * NC + c
    base = w * CPT

    _zero_fill_vmem(zeros_v, 16, HID)
    _zero_acc_slice(zeros_v, acc, s)

    isems = (isem0, isem1)
    rbufs = (rows0, rows1)
    gsems = (sem0, sem1)

    def idx_load(b, k):
        pltpu.async_copy(src_hbm.at[pl.ds(base + k * SLAB, SLAB)],
                         src_v.at[b], isems[b])
        pltpu.async_copy(dst_hbm.at[pl.ds(base + k * SLAB, SLAB)],
                         dst_v.at[b], isems[b])

    def idx_wait(b):
        pltpu.make_async_copy(src_hbm.at[pl.ds(base, SLAB)],
                              src_v.at[b], isems[b]).wait()
        pltpu.make_async_copy(dst_hbm.at[pl.ds(base, SLAB)],
                              dst_v.at[b], isems[b]).wait()

    idx_load(0, 0)

    plsc.subcore_barrier()

    def process_slab(b, k, prefetch_k):
        idx_wait(b)
        ob = 1 - b
        if prefetch_k is not None:
            idx_load(ob, prefetch_k)
        pltpu.async_copy(y_hbm.at[src_v.at[b, 0]], rbufs[0], gsems[0])
        for j in range(SLAB):
            rb = j % 2
            if j + 1 < SLAB:
                pltpu.async_copy(y_hbm.at[src_v.at[b, j + 1]],
                                 rbufs[1 - rb], gsems[1 - rb])
            pltpu.make_async_copy(y_hbm.at[src_v.at[b, 0]],
                                  rbufs[rb], gsems[rb]).wait()
            pltpu.sync_copy(rbufs[rb], acc.at[dst_v.at[b, j]], add=True)

    @pl.loop(0, NSLAB // 2)
    def _(t):
        k = 2 * t
        process_slab(0, k, k + 1)
        nxt = k + 2

        def load_next():
            idx_load(0, nxt)

        pl.when(nxt < NSLAB)(load_next)
        process_slab(1, k + 1, None)

    plsc.subcore_barrier()

    pltpu.sync_copy(
        acc.at[pl.ds(s * ROWS_PER_TILE, ROWS_PER_TILE)],
        out_hbm.at[c, pl.ds(s * ROWS_PER_TILE, ROWS_PER_TILE)],
    )


def _dinv_from_deg(degp_ref):
    deg = degp_ref[0, :N, 0:1] + degp_ref[1, :N, 0:1] + 1.0
    return lax.rsqrt(deg)


def _tc0_body(x_ref, w1_ref, xw_ref):
    xw_ref[...] = jnp.dot(x_ref[...], w1_ref[...],
                          preferred_element_type=jnp.float32)


def _tc1_body(xw_ref, degp_ref, y_ref):
    y_ref[...] = xw_ref[...] * _dinv_from_deg(degp_ref)


def _tc2_body(y_ref, aggp_ref, degp_ref, w2_ref, b1_ref, y2_ref):
    dinv = _dinv_from_deg(degp_ref)
    z = aggp_ref[0, :N, :] + aggp_ref[1, :N, :] + y_ref[...]
    h = jnp.maximum(dinv * z + b1_ref[...], 0.0)
    y2_ref[...] = jnp.dot(h, w2_ref[...], preferred_element_type=jnp.float32) * dinv


def _tc3_body(y_ref, aggp_ref, degp_ref, w3_ref, b2_ref, b3_ref, out_ref):
    dinv = _dinv_from_deg(degp_ref)
    z = aggp_ref[0, :N, :] + aggp_ref[1, :N, :] + y_ref[...]
    h = jnp.maximum(dinv * z + b2_ref[...], 0.0)
    logits = jnp.dot(h, w3_ref[...], preferred_element_type=jnp.float32) + b3_ref[...]
    m = jnp.max(logits, axis=1, keepdims=True)
    e = jnp.exp(logits - m)
    lse = jnp.log(jnp.sum(e, axis=1, keepdims=True)) + m
    out_ref[...] = logits - lse


def kernel(x, edge_index, W1, b1, W2, b2, W3, b3):
    pad_iota = jnp.arange(E_PAD, dtype=jnp.int32)
    pad_src = pad_iota % N
    pad_dst = N + pad_iota % (N_PAD - N)
    src = jnp.concatenate([edge_index[0], pad_src]).reshape(NCHUNK, CHUNK)
    dst = jnp.concatenate([edge_index[1], pad_dst]).reshape(NCHUNK, CHUNK)

    degp = _deg_kernel(dst)
    xw1 = pl.pallas_call(
        _tc0_body,
        out_shape=jax.ShapeDtypeStruct((N, D_IN), jnp.float32),
    )(x.astype(jnp.float32), W1)

    y1 = pl.pallas_call(
        _tc1_body,
        out_shape=jax.ShapeDtypeStruct((N, D_IN), jnp.float32),
    )(xw1, degp)

    agg1 = _scatter_kernel(src, dst, y1)

    y2 = pl.pallas_call(
        _tc2_body,
        out_shape=jax.ShapeDtypeStruct((N, HID), jnp.float32),
    )(y1, agg1, degp, W2, b1.reshape(1, HID))

    agg2 = _scatter_kernel(src, dst, y2)

    out = pl.pallas_call(
        _tc3_body,
        out_shape=jax.ShapeDtypeStruct((N, OUT), jnp.float32),
    )(y2, agg2, degp, W3, b2.reshape(1, HID), b3.reshape(1, OUT))

    return out


# speedup vs baseline: 3.2645x; 1.0799x over previous
import functools

import jax
import jax.numpy as jnp
from jax import lax
from jax.experimental import pallas as pl
from jax.experimental.pallas import tpu as pltpu
from jax.experimental.pallas import tpu_sc as plsc

N = 10000
E = 320000
D_IN = 128
HID = 128
OUT = 64

NC = 2
NS = 16
NW = NC * NS
CHUNK = 128
SLAB = 8
NSLAB = 10
CPT = SLAB * NSLAB
NCHUNK = NW * CPT
E_PAD = NCHUNK * CHUNK - E
N_PAD = 10240
ROWS_PER_TILE = N_PAD // NS

_mesh = plsc.VectorSubcoreMesh(core_axis_name="c", subcore_axis_name="s")


def _zero_fill_vmem(buf, rows, width):
    zero16 = jnp.zeros((16,), jnp.float32)

    @pl.loop(0, rows)
    def _(i):
        @pl.loop(0, width // 16)
        def _(j):
            buf[i, pl.ds(j * 16, 16)] = zero16


@functools.partial(
    pl.kernel,
    out_type=jax.ShapeDtypeStruct((NC, N_PAD, 16), jnp.float32),
    mesh=_mesh,
    scratch_types=[
        pltpu.VMEM((CPT // 2, CHUNK), jnp.int32),
        pltpu.VMEM((CHUNK, 16), jnp.float32),
        pltpu.VMEM((16, 16), jnp.float32),
        pltpu.VMEM_SHARED((N_PAD, 16), jnp.float32),
    ],
)
def _deg_kernel(dst_hbm, out_hbm, dst_v, ones_v, zeros_v, acc):
    c = lax.axis_index("c")
    s = lax.axis_index("s")
    w = s * NC + c

    one16 = jnp.ones((16,), jnp.float32)

    @pl.loop(0, CHUNK)
    def _(i):
        ones_v[i, pl.ds(0, 16)] = one16

    _zero_fill_vmem(zeros_v, 16, 16)

    @pl.loop(0, ROWS_PER_TILE // 16)
    def _(j):
        pltpu.sync_copy(zeros_v, acc.at[pl.ds(s * ROWS_PER_TILE + j * 16, 16)])

    plsc.subcore_barrier()

    for half in range(2):
        pltpu.sync_copy(
            dst_hbm.at[pl.ds(w * CPT + half * (CPT // 2), CPT // 2)], dst_v)

        @pl.loop(0, CPT // 2)
        def _(j):
            pltpu.sync_copy(ones_v, acc.at[dst_v.at[j]], add=True)

    plsc.subcore_barrier()

    pltpu.sync_copy(
        acc.at[pl.ds(s * ROWS_PER_TILE, ROWS_PER_TILE)],
        out_hbm.at[c, pl.ds(s * ROWS_PER_TILE, ROWS_PER_TILE)],
    )


@functools.partial(
    pl.kernel,
    out_type=jax.ShapeDtypeStruct((NC, N_PAD, HID), jnp.float32),
    mesh=_mesh,
    scratch_types=[
        pltpu.VMEM((2, SLAB, CHUNK), jnp.int32),
        pltpu.VMEM((2, SLAB, CHUNK), jnp.int32),
        pltpu.VMEM((CHUNK, HID), jnp.float32),
        pltpu.VMEM((CHUNK, HID), jnp.float32),
        pltpu.VMEM_SHARED((N_PAD, HID), jnp.float32),
        pltpu.SemaphoreType.DMA,
        pltpu.SemaphoreType.DMA,
        pltpu.SemaphoreType.DMA,
        pltpu.SemaphoreType.DMA,
    ],
)
def _scatter_kernel(src_hbm, dst_hbm, y_hbm, out_hbm,
                    src_v, dst_v, r0, r1, acc,
                    g0, g1, i0, i1):
    c = lax.axis_index("c")
    s = lax.axis_index("s")
    w = s * NC + c
    base = w * CPT

    rbufs = (r0, r1)
    gsems = (g0, g1)
    isems = (i0, i1)

    def idx_load(b, k):
        pltpu.async_copy(src_hbm.at[pl.ds(base + k * SLAB, SLAB)],
                         src_v.at[b], isems[b])
        pltpu.async_copy(dst_hbm.at[pl.ds(base + k * SLAB, SLAB)],
                         dst_v.at[b], isems[b])

    def idx_wait(b):
        pltpu.make_async_copy(src_hbm.at[pl.ds(base, SLAB)],
                              src_v.at[b], isems[b]).wait()
        pltpu.make_async_copy(dst_hbm.at[pl.ds(base, SLAB)],
                              dst_v.at[b], isems[b]).wait()

    def issue_gather(q):
        ss, jj = divmod(q, SLAB)
        r = q % 2
        pltpu.async_copy(y_hbm.at[src_v.at[ss % 2, jj]], rbufs[r], gsems[r])

    idx_load(0, 0)
    idx_load(1, 1)

    _zero_fill_vmem(r0, CHUNK, HID)
    for i in range(ROWS_PER_TILE // CHUNK):
        pltpu.sync_copy(
            r0, acc.at[pl.ds(s * ROWS_PER_TILE + i * CHUNK, CHUNK)])

    plsc.subcore_barrier()

    idx_wait(0)
    issue_gather(0)
    for q in range(CPT):
        ss, jj = divmod(q, SLAB)
        r = q % 2
        if q + 1 < CPT:
            s2, j2 = divmod(q + 1, SLAB)
            if j2 == 0:
                idx_wait(s2 % 2)
            issue_gather(q + 1)
        pltpu.make_async_copy(y_hbm.at[src_v.at[0, 0]],
                              rbufs[r], gsems[r]).wait()
        pltpu.sync_copy(rbufs[r], acc.at[dst_v.at[ss % 2, jj]], add=True)
        if jj == SLAB - 1 and ss + 2 < NSLAB:
            idx_load(ss % 2, ss + 2)

    plsc.subcore_barrier()

    pltpu.sync_copy(
        acc.at[pl.ds(s * ROWS_PER_TILE, ROWS_PER_TILE)],
        out_hbm.at[c, pl.ds(s * ROWS_PER_TILE, ROWS_PER_TILE)],
    )


def _dinv_from_deg(degp_ref):
    deg = degp_ref[0, :N, 0:1] + degp_ref[1, :N, 0:1] + 1.0
    return lax.rsqrt(deg)


def _tc0_body(x_ref, w1_ref, xw_ref):
    xw_ref[...] = jnp.dot(x_ref[...], w1_ref[...],
                          preferred_element_type=jnp.float32)


def _tc1_body(xw_ref, degp_ref, y_ref):
    y_ref[...] = xw_ref[...] * _dinv_from_deg(degp_ref)


def _tc2_body(y_ref, aggp_ref, degp_ref, w2_ref, b1_ref, y2_ref):
    dinv = _dinv_from_deg(degp_ref)
    z = aggp_ref[0, :N, :] + aggp_ref[1, :N, :] + y_ref[...]
    h = jnp.maximum(dinv * z + b1_ref[...], 0.0)
    y2_ref[...] = jnp.dot(h, w2_ref[...], preferred_element_type=jnp.float32) * dinv


def _tc3_body(y_ref, aggp_ref, degp_ref, w3_ref, b2_ref, b3_ref, out_ref):
    dinv = _dinv_from_deg(degp_ref)
    z = aggp_ref[0, :N, :] + aggp_ref[1, :N, :] + y_ref[...]
    h = jnp.maximum(dinv * z + b2_ref[...], 0.0)
    logits = jnp.dot(h, w3_ref[...], preferred_element_type=jnp.float32) + b3_ref[...]
    m = jnp.max(logits, axis=1, keepdims=True)
    e = jnp.exp(logits - m)
    lse = jnp.log(jnp.sum(e, axis=1, keepdims=True)) + m
    out_ref[...] = logits - lse


def kernel(x, edge_index, W1, b1, W2, b2, W3, b3):
    pad_iota = jnp.arange(E_PAD, dtype=jnp.int32)
    pad_src = pad_iota % N
    pad_dst = N + pad_iota % (N_PAD - N)
    src = jnp.concatenate([edge_index[0], pad_src]).reshape(NCHUNK, CHUNK)
    dst = jnp.concatenate([edge_index[1], pad_dst]).reshape(NCHUNK, CHUNK)

    degp = _deg_kernel(dst)
    xw1 = pl.pallas_call(
        _tc0_body,
        out_shape=jax.ShapeDtypeStruct((N, D_IN), jnp.float32),
    )(x.astype(jnp.float32), W1)

    y1 = pl.pallas_call(
        _tc1_body,
        out_shape=jax.ShapeDtypeStruct((N, D_IN), jnp.float32),
    )(xw1, degp)

    agg1 = _scatter_kernel(src, dst, y1)

    y2 = pl.pallas_call(
        _tc2_body,
        out_shape=jax.ShapeDtypeStruct((N, HID), jnp.float32),
    )(y1, agg1, degp, W2, b1.reshape(1, HID))

    agg2 = _scatter_kernel(src, dst, y2)

    out = pl.pallas_call(
        _tc3_body,
        out_shape=jax.ShapeDtypeStruct((N, OUT), jnp.float32),
    )(y2, agg2, degp, W3, b2.reshape(1, HID), b3.reshape(1, OUT))

    return out
